# initial kernel scaffold (unmeasured)
import jax
import jax.numpy as jnp
from jax import lax
from jax.experimental import pallas as pl
from jax.experimental.pallas import tpu as pltpu

N_DEV = 4
NT = 2048


def kernel(x, w_mat):
    m, k_sh = x.shape
    _, n = w_mat.shape
    m_blk = m // N_DEV
    nb = n // NT

    dot = lambda a, b: jnp.dot(
        a, b, preferred_element_type=jnp.float32,
        precision=lax.Precision.HIGHEST,
    )

    def body(x_hbm, w_hbm, out_hbm, xb, wb, comm,
             sem_x, sem_w, sem_out,
             send_sems, recv_sems, credit_sem,
             amax_send, amax_all, amax_ssems, amax_rsems, amax_smem):
        d = lax.axis_index("i")
        left = (d - 1) % N_DEV
        right = (d + 1) % N_DEV

        amax_smem[0] = 0.0
        amax_all[...] = jnp.zeros_like(amax_all)

        barrier_sem = pltpu.get_barrier_semaphore()
        for nbr in [left, right]:
            pl.semaphore_signal(
                barrier_sem, inc=1,
                device_id=(nbr,), device_id_type=pl.DeviceIdType.MESH,
            )
        pl.semaphore_wait(barrier_sem, 2)

        def load_x_block(c):
            cp = pltpu.make_async_copy(
                x_hbm.at[pl.ds(c * m_blk, m_blk), :], xb, sem_x)
            cp.start()
            cp.wait()

        for j in range(nb):
            ncols = pl.ds(j * NT, NT)
            cp = pltpu.make_async_copy(w_hbm.at[:, ncols], wb, sem_w)
            cp.start()
            cp.wait()

            for h in range(N_DEV - 1):
                send_slot = h % 2
                recv_slot = (h + 1) % 2
                c = (d - 1 - h) % N_DEV
                load_x_block(c)
                if h == 0:
                    comm[send_slot] = dot(xb[...], wb[...])
                else:
                    comm[send_slot] = comm[send_slot] + dot(xb[...], wb[...])

                if h >= 1 or j >= 1:
                    pl.semaphore_wait(credit_sem, 1)
                rdma = pltpu.make_async_remote_copy(
                    src_ref=comm.at[send_slot],
                    dst_ref=comm.at[recv_slot],
                    send_sem=send_sems.at[send_slot],
                    recv_sem=recv_sems.at[recv_slot],
                    device_id=(right,),
                    device_id_type=pl.DeviceIdType.MESH,
                )
                rdma.start()
                rdma.wait()
                pl.semaphore_signal(
                    credit_sem, inc=1,
                    device_id=(left,), device_id_type=pl.DeviceIdType.MESH,
                )

            load_x_block(d)
            y = jnp.maximum(comm[1] + dot(xb[...], wb[...]), 0.0)
            amax_smem[0] = jnp.maximum(amax_smem[0], jnp.max(y))
            comm[0] = y
            cp = pltpu.make_async_copy(comm.at[0], out_hbm.at[:, ncols], sem_out)
            cp.start()
            cp.wait()
            if j < nb - 1:
                pl.semaphore_signal(
                    credit_sem, inc=1,
                    device_id=(left,), device_id_type=pl.DeviceIdType.MESH,
                )

        amax_send[...] = jnp.full((1, 128), amax_smem[0], jnp.float32)
        sends = []
        for off in (1, 2, 3):
            p = (d + off) % N_DEV
            s = pltpu.make_async_remote_copy(
                src_ref=amax_send,
                dst_ref=amax_all.at[d],
                send_sem=amax_ssems.at[off],
                recv_sem=amax_rsems.at[d],
                device_id=(p,),
                device_id_type=pl.DeviceIdType.MESH,
            )
            s.start()
            sends.append(s)
        for off in (1, 2, 3):
            p = (d + off) % N_DEV
            r = pltpu.make_async_remote_copy(
                src_ref=amax_send,
                dst_ref=amax_all.at[p],
                send_sem=amax_ssems.at[0],
                recv_sem=amax_rsems.at[p],
                device_id=(p,),
                device_id_type=pl.DeviceIdType.MESH,
            )
            r.wait_recv()
        for s in sends:
            s.wait_send()

        g = jnp.maximum(jnp.maximum(amax_smem[0], jnp.max(amax_all[...])),
                        1e-20)
        scale = g / 127.0
        recip = 127.0 / g

        for j in range(nb):
            ncols = pl.ds(j * NT, NT)
            cp = pltpu.make_async_copy(out_hbm.at[:, ncols], comm.at[0], sem_out)
            cp.start()
            cp.wait()
            q = jnp.clip(jnp.round(comm[0] * recip), -127.0, 127.0)
            comm[1] = q * scale
            cp = pltpu.make_async_copy(comm.at[1], out_hbm.at[:, ncols], sem_out)
            cp.start()
            cp.wait()

    return pl.pallas_call(
        body,
        out_shape=jax.ShapeDtypeStruct((m_blk, n), jnp.float32),
        in_specs=[
            pl.BlockSpec(memory_space=pltpu.ANY),
            pl.BlockSpec(memory_space=pltpu.ANY),
        ],
        out_specs=pl.BlockSpec(memory_space=pltpu.ANY),
        scratch_shapes=[
            pltpu.VMEM((m_blk, k_sh), jnp.float32),
            pltpu.VMEM((k_sh, NT), jnp.float32),
            pltpu.VMEM((2, m_blk, NT), jnp.float32),
            pltpu.SemaphoreType.DMA,
            pltpu.SemaphoreType.DMA,
            pltpu.SemaphoreType.DMA,
            pltpu.SemaphoreType.DMA((2,)),
            pltpu.SemaphoreType.DMA((2,)),
            pltpu.SemaphoreType.REGULAR,
            pltpu.VMEM((1, 128), jnp.float32),
            pltpu.VMEM((N_DEV, 1, 128), jnp.float32),
            pltpu.SemaphoreType.DMA((N_DEV,)),
            pltpu.SemaphoreType.DMA((N_DEV,)),
            pltpu.SMEM((1,), jnp.float32),
        ],
        compiler_params=pltpu.CompilerParams(collective_id=0),
    )(x, w_mat)


# baseline (device time: 1746274 ns/iter reference)
import jax
import jax.numpy as jnp
from jax import lax
from jax.experimental import pallas as pl
from jax.experimental.pallas import tpu as pltpu

N_DEV = 4
NT = 1024


def kernel(x, w_mat):
    m, k_sh = x.shape
    _, n = w_mat.shape
    m_blk = m // N_DEV
    nb = n // NT

    dot = lambda a, b: jnp.dot(
        a, b, preferred_element_type=jnp.float32,
        precision=lax.Precision.HIGHEST,
    )

    def body(x_hbm, w_hbm, out_hbm, xb, wb, comm,
             sem_x, sem_w, sem_out,
             send_sems, recv_sems, credit_sem,
             amax_send, amax_all, amax_ssems, amax_rsems, amax_smem):
        d = lax.axis_index("i")
        left = (d - 1) % N_DEV
        right = (d + 1) % N_DEV

        amax_smem[0] = 0.0
        amax_all[...] = jnp.zeros((N_DEV, 1, 128), jnp.float32)

        barrier_sem = pltpu.get_barrier_semaphore()
        for nbr in [left, right]:
            pl.semaphore_signal(
                barrier_sem, inc=1,
                device_id=(nbr,), device_id_type=pl.DeviceIdType.MESH,
            )
        pl.semaphore_wait(barrier_sem, 2)

        def load_x_block(c):
            cp = pltpu.make_async_copy(
                x_hbm.at[pl.ds(c * m_blk, m_blk), :], xb, sem_x)
            cp.start()
            cp.wait()

        def signal_credit():
            pl.semaphore_signal(
                credit_sem, inc=1,
                device_id=(left,), device_id_type=pl.DeviceIdType.MESH,
            )

        def subring(j, carry):
            ncols = pl.ds(j * NT, NT)
            cp = pltpu.make_async_copy(w_hbm.at[:, ncols], wb, sem_w)
            cp.start()
            cp.wait()

            for h in range(N_DEV - 1):
                send_slot = h % 2
                recv_slot = (h + 1) % 2
                c = (d - 1 - h) % N_DEV
                load_x_block(c)
                if h == 0:
                    comm[send_slot] = dot(xb[...], wb[...])
                else:
                    comm[send_slot] = comm[send_slot] + dot(xb[...], wb[...])

                if h >= 1:
                    pl.semaphore_wait(credit_sem, 1)
                else:
                    @pl.when(j >= 1)
                    def _():
                        pl.semaphore_wait(credit_sem, 1)
                rdma = pltpu.make_async_remote_copy(
                    src_ref=comm.at[send_slot],
                    dst_ref=comm.at[recv_slot],
                    send_sem=send_sems.at[send_slot],
                    recv_sem=recv_sems.at[recv_slot],
                    device_id=(right,),
                    device_id_type=pl.DeviceIdType.MESH,
                )
                rdma.start()
                rdma.wait()
                if h < N_DEV - 2:
                    signal_credit()

            load_x_block(d)
            y = jnp.maximum(comm[1] + dot(xb[...], wb[...]), 0.0)
            amax_smem[0] = jnp.maximum(amax_smem[0], jnp.max(y))
            comm[0] = y
            cp = pltpu.make_async_copy(comm.at[0], out_hbm.at[:, ncols], sem_out)
            cp.start()
            cp.wait()
            @pl.when(j < nb - 1)
            def _():
                signal_credit()
            return carry

        lax.fori_loop(0, nb, subring, 0)

        amax_send[...] = jnp.full((1, 128), amax_smem[0], jnp.float32)
        sends = []
        for off in (1, 2, 3):
            p = (d + off) % N_DEV
            s = pltpu.make_async_remote_copy(
                src_ref=amax_send,
                dst_ref=amax_all.at[d],
                send_sem=amax_ssems.at[off],
                recv_sem=amax_rsems.at[d],
                device_id=(p,),
                device_id_type=pl.DeviceIdType.MESH,
            )
            s.start()
            sends.append(s)
        for off in (1, 2, 3):
            p = (d + off) % N_DEV
            r = pltpu.make_async_remote_copy(
                src_ref=amax_send,
                dst_ref=amax_all.at[p],
                send_sem=amax_ssems.at[0],
                recv_sem=amax_rsems.at[p],
                device_id=(p,),
                device_id_type=pl.DeviceIdType.MESH,
            )
            r.wait_recv()
        for s in sends:
            s.wait_send()

        g = jnp.maximum(jnp.maximum(amax_smem[0], jnp.max(amax_all[...])),
                        1e-20)
        scale = g / 127.0
        recip = 127.0 / g

        def quant_tile(j, carry):
            ncols = pl.ds(j * NT, NT)
            cp = pltpu.make_async_copy(out_hbm.at[:, ncols], comm.at[0], sem_out)
            cp.start()
            cp.wait()
            q = jnp.clip(jnp.round(comm[0] * recip), -127.0, 127.0)
            comm[1] = q * scale
            cp = pltpu.make_async_copy(comm.at[1], out_hbm.at[:, ncols], sem_out)
            cp.start()
            cp.wait()
            return carry

        lax.fori_loop(0, nb, quant_tile, 0)

    return pl.pallas_call(
        body,
        out_shape=jax.ShapeDtypeStruct((m_blk, n), jnp.float32),
        in_specs=[
            pl.BlockSpec(memory_space=pl.ANY),
            pl.BlockSpec(memory_space=pl.ANY),
        ],
        out_specs=pl.BlockSpec(memory_space=pl.ANY),
        scratch_shapes=[
            pltpu.VMEM((m_blk, k_sh), jnp.float32),
            pltpu.VMEM((k_sh, NT), jnp.float32),
            pltpu.VMEM((2, m_blk, NT), jnp.float32),
            pltpu.SemaphoreType.DMA,
            pltpu.SemaphoreType.DMA,
            pltpu.SemaphoreType.DMA,
            pltpu.SemaphoreType.DMA((2,)),
            pltpu.SemaphoreType.DMA((2,)),
            pltpu.SemaphoreType.REGULAR,
            pltpu.VMEM((1, 128), jnp.float32),
            pltpu.VMEM((N_DEV, 1, 128), jnp.float32),
            pltpu.SemaphoreType.DMA((N_DEV,)),
            pltpu.SemaphoreType.DMA((N_DEV,)),
            pltpu.SMEM((1,), jnp.float32),
        ],
        compiler_params=pltpu.CompilerParams(
            collective_id=0,
            vmem_limit_bytes=44 * 1024 * 1024,
        ),
    )(x, w_mat)


# device time: 744366 ns/iter; 2.3460x vs baseline; 2.3460x over previous
import jax
import jax.numpy as jnp
from jax import lax
from jax.experimental import pallas as pl
from jax.experimental.pallas import tpu as pltpu

N_DEV = 4
NT = 1024


def kernel(x, w_mat):
    m, k_sh = x.shape
    _, n = w_mat.shape
    m_blk = m // N_DEV
    half = n // 2
    nb = half // NT

    bdot = lambda a, b: jnp.dot(a, b, preferred_element_type=jnp.float32)

    def body(x_hbm, w_hbm, out_hbm, stage, xhi, xlo, wahi, walo, wbhi, wblo,
             tmpa, tmpb, comm_a, comm_b,
             sem_x, sem_w, sem_out,
             send_a, recv_a, send_b, recv_b, credit_a, credit_b,
             amax_send, amax_all, amax_ssems, amax_rsems, amax_smem):
        d = lax.axis_index("i")
        left = (d - 1) % N_DEV
        right = (d + 1) % N_DEV

        amax_smem[0] = 0.0
        amax_all[...] = jnp.zeros((N_DEV, 1, 128), jnp.float32)

        barrier_sem = pltpu.get_barrier_semaphore()
        for nbr in [left, right]:
            pl.semaphore_signal(
                barrier_sem, inc=1,
                device_id=(nbr,), device_id_type=pl.DeviceIdType.MESH,
            )
        pl.semaphore_wait(barrier_sem, 2)

        def split(src_f32, hi, lo):
            h = src_f32.astype(jnp.bfloat16)
            hi[...] = h
            lo[...] = (src_f32 - h.astype(jnp.float32)).astype(jnp.bfloat16)

        def load_x(c):
            cp = pltpu.make_async_copy(
                x_hbm.at[pl.ds(c * m_blk, m_blk), :], stage, sem_x)
            cp.start()
            cp.wait()
            split(stage[...], xhi, xlo)

        def load_w(col_start, whi, wlo):
            cp = pltpu.make_async_copy(
                w_hbm.at[:, pl.ds(col_start, NT)], stage, sem_w)
            cp.start()
            cp.wait()
            split(stage[...], whi, wlo)

        def mm(acc, whi, wlo):
            acc[...] = bdot(xhi[...], whi[...])
            acc[...] = acc[...] + bdot(xhi[...], wlo[...])
            acc[...] = acc[...] + bdot(xlo[...], whi[...])

        def rdma_a(send_slot):
            return pltpu.make_async_remote_copy(
                src_ref=comm_a.at[send_slot],
                dst_ref=comm_a.at[(send_slot + 1) % 2],
                send_sem=send_a.at[send_slot],
                recv_sem=recv_a.at[(send_slot + 1) % 2],
                device_id=(right,), device_id_type=pl.DeviceIdType.MESH,
            )

        def rdma_b(send_slot):
            return pltpu.make_async_remote_copy(
                src_ref=comm_b.at[send_slot],
                dst_ref=comm_b.at[(send_slot + 1) % 2],
                send_sem=send_b.at[send_slot],
                recv_sem=recv_b.at[(send_slot + 1) % 2],
                device_id=(left,), device_id_type=pl.DeviceIdType.MESH,
            )

        def credit(sem, to):
            pl.semaphore_signal(
                sem, inc=1, device_id=(to,),
                device_id_type=pl.DeviceIdType.MESH,
            )

        def subring(j, carry):
            cols_a = j * NT
            cols_b = half + j * NT
            load_w(cols_a, wahi, walo)
            load_w(cols_b, wbhi, wblo)

            load_x((d - 1) % N_DEV)
            mm(comm_a.at[0], wahi, walo)

            @pl.when(j >= 1)
            def _():
                pl.semaphore_wait(credit_a, 1)
            a0 = rdma_a(0)
            a0.start()

            load_x((d + 1) % N_DEV)
            mm(comm_b.at[0], wbhi, wblo)

            @pl.when(j >= 1)
            def _():
                pl.semaphore_wait(credit_b, 1)
            b0 = rdma_b(0)
            b0.start()

            load_x((d - 2) % N_DEV)
            mm(tmpa, wahi, walo)
            load_x((d + 2) % N_DEV)
            mm(tmpb, wbhi, wblo)

            a0.wait()
            credit(credit_a, left)
            comm_a[1] = comm_a[1] + tmpa[...]
            pl.semaphore_wait(credit_a, 1)
            a1 = rdma_a(1)
            a1.start()

            b0.wait()
            credit(credit_b, right)
            comm_b[1] = comm_b[1] + tmpb[...]
            pl.semaphore_wait(credit_b, 1)
            b1 = rdma_b(1)
            b1.start()

            load_x((d - 3) % N_DEV)
            mm(tmpa, wahi, walo)
            load_x((d + 3) % N_DEV)
            mm(tmpb, wbhi, wblo)

            a1.wait()
            credit(credit_a, left)
            comm_a[0] = comm_a[0] + tmpa[...]
            pl.semaphore_wait(credit_a, 1)
            a2 = rdma_a(0)
            a2.start()

            b1.wait()
            credit(credit_b, right)
            comm_b[0] = comm_b[0] + tmpb[...]
            pl.semaphore_wait(credit_b, 1)
            b2 = rdma_b(0)
            b2.start()

            load_x(d)
            mm(tmpa, wahi, walo)
            mm(tmpb, wbhi, wblo)

            a2.wait()
            ya = jnp.maximum(comm_a[1] + tmpa[...], 0.0)
            amax_smem[0] = jnp.maximum(amax_smem[0], jnp.max(ya))
            comm_a[0] = ya
            cp = pltpu.make_async_copy(
                comm_a.at[0], out_hbm.at[:, pl.ds(cols_a, NT)], sem_out)
            cp.start()
            cp.wait()

            b2.wait()
            yb = jnp.maximum(comm_b[1] + tmpb[...], 0.0)
            amax_smem[0] = jnp.maximum(amax_smem[0], jnp.max(yb))
            comm_b[0] = yb
            cp = pltpu.make_async_copy(
                comm_b.at[0], out_hbm.at[:, pl.ds(cols_b, NT)], sem_out)
            cp.start()
            cp.wait()

            @pl.when(j < nb - 1)
            def _():
                credit(credit_a, left)
                credit(credit_b, right)
            return carry

        lax.fori_loop(0, nb, subring, 0)

        amax_send[...] = jnp.full((1, 128), amax_smem[0], jnp.float32)
        sends = []
        for off in (1, 2, 3):
            p = (d + off) % N_DEV
            s = pltpu.make_async_remote_copy(
                src_ref=amax_send,
                dst_ref=amax_all.at[d],
                send_sem=amax_ssems.at[off],
                recv_sem=amax_rsems.at[d],
                device_id=(p,), device_id_type=pl.DeviceIdType.MESH,
            )
            s.start()
            sends.append(s)
        for off in (1, 2, 3):
            p = (d + off) % N_DEV
            r = pltpu.make_async_remote_copy(
                src_ref=amax_send,
                dst_ref=amax_all.at[p],
                send_sem=amax_ssems.at[0],
                recv_sem=amax_rsems.at[p],
                device_id=(p,), device_id_type=pl.DeviceIdType.MESH,
            )
            r.wait_recv()
        for s in sends:
            s.wait_send()

        g = jnp.maximum(jnp.maximum(amax_smem[0], jnp.max(amax_all[...])),
                        1e-20)
        scale = g / 127.0
        recip = 127.0 / g

        def quant_tile(j, carry):
            ncols = pl.ds(j * NT, NT)
            cp = pltpu.make_async_copy(out_hbm.at[:, ncols], comm_a.at[0],
                                       sem_out)
            cp.start()
            cp.wait()
            q = jnp.clip(jnp.round(comm_a[0] * recip), -127.0, 127.0)
            comm_a[1] = q * scale
            cp = pltpu.make_async_copy(comm_a.at[1], out_hbm.at[:, ncols],
                                       sem_out)
            cp.start()
            cp.wait()
            return carry

        lax.fori_loop(0, n // NT, quant_tile, 0)

    return pl.pallas_call(
        body,
        out_shape=jax.ShapeDtypeStruct((m_blk, n), jnp.float32),
        in_specs=[
            pl.BlockSpec(memory_space=pl.ANY),
            pl.BlockSpec(memory_space=pl.ANY),
        ],
        out_specs=pl.BlockSpec(memory_space=pl.ANY),
        scratch_shapes=[
            pltpu.VMEM((m_blk, k_sh), jnp.float32),
            pltpu.VMEM((m_blk, k_sh), jnp.bfloat16),
            pltpu.VMEM((m_blk, k_sh), jnp.bfloat16),
            pltpu.VMEM((k_sh, NT), jnp.bfloat16),
            pltpu.VMEM((k_sh, NT), jnp.bfloat16),
            pltpu.VMEM((k_sh, NT), jnp.bfloat16),
            pltpu.VMEM((k_sh, NT), jnp.bfloat16),
            pltpu.VMEM((m_blk, NT), jnp.float32),
            pltpu.VMEM((m_blk, NT), jnp.float32),
            pltpu.VMEM((2, m_blk, NT), jnp.float32),
            pltpu.VMEM((2, m_blk, NT), jnp.float32),
            pltpu.SemaphoreType.DMA,
            pltpu.SemaphoreType.DMA,
            pltpu.SemaphoreType.DMA,
            pltpu.SemaphoreType.DMA((2,)),
            pltpu.SemaphoreType.DMA((2,)),
            pltpu.SemaphoreType.DMA((2,)),
            pltpu.SemaphoreType.DMA((2,)),
            pltpu.SemaphoreType.REGULAR,
            pltpu.SemaphoreType.REGULAR,
            pltpu.VMEM((1, 128), jnp.float32),
            pltpu.VMEM((N_DEV, 1, 128), jnp.float32),
            pltpu.SemaphoreType.DMA((N_DEV,)),
            pltpu.SemaphoreType.DMA((N_DEV,)),
            pltpu.SMEM((1,), jnp.float32),
        ],
        compiler_params=pltpu.CompilerParams(
            collective_id=0,
            vmem_limit_bytes=58 * 1024 * 1024,
        ),
    )(x, w_mat)


# device time: 477068 ns/iter; 3.6604x vs baseline; 1.5603x over previous
import jax
import jax.numpy as jnp
from jax import lax
from jax.experimental import pallas as pl
from jax.experimental.pallas import tpu as pltpu

N_DEV = 4
NT = 1024


def kernel(x, w_mat):
    m, k_sh = x.shape
    _, n = w_mat.shape
    m_blk = m // N_DEV
    half = n // 2
    nb = half // NT

    bdot = lambda a, b: jnp.dot(a, b, preferred_element_type=jnp.float32)

    def body(x_hbm, w_hbm, out_hbm, stage, xhi, xlo, wahi, walo, wbhi, wblo,
             tmpa, tmpb, comm_a, comm_b,
             sem_x, sem_w, sem_out,
             send_a, recv_a, send_b, recv_b, credit_a, credit_b,
             amax_send, amax_all, amax_ssems, amax_rsems, amax_smem):
        d = lax.axis_index("i")
        left = (d - 1) % N_DEV
        right = (d + 1) % N_DEV

        amax_smem[0] = 0.0
        amax_all[...] = jnp.zeros((N_DEV, 1, 128), jnp.float32)

        barrier_sem = pltpu.get_barrier_semaphore()
        for nbr in [left, right]:
            pl.semaphore_signal(
                barrier_sem, inc=1,
                device_id=(nbr,), device_id_type=pl.DeviceIdType.MESH,
            )
        pl.semaphore_wait(barrier_sem, 2)

        def split(src_f32, hi, lo):
            h = src_f32.astype(jnp.bfloat16)
            hi[...] = h
            lo[...] = (src_f32 - h.astype(jnp.float32)).astype(jnp.bfloat16)

        def load_x(c):
            cp = pltpu.make_async_copy(
                x_hbm.at[pl.ds(c * m_blk, m_blk), :], stage, sem_x)
            cp.start()
            cp.wait()
            split(stage[...], xhi, xlo)

        def load_w(col_start, whi, wlo):
            cp = pltpu.make_async_copy(
                w_hbm.at[:, pl.ds(col_start, NT)], stage, sem_w)
            cp.start()
            cp.wait()
            split(stage[...], whi, wlo)

        def mm(acc, whi, wlo):
            acc[...] = bdot(xhi[...], whi[...])
            acc[...] = acc[...] + bdot(xhi[...], wlo[...])
            acc[...] = acc[...] + bdot(xlo[...], whi[...])

        def rdma_a(send_slot):
            return pltpu.make_async_remote_copy(
                src_ref=comm_a.at[send_slot],
                dst_ref=comm_a.at[(send_slot + 1) % 2],
                send_sem=send_a.at[send_slot],
                recv_sem=recv_a.at[(send_slot + 1) % 2],
                device_id=(right,), device_id_type=pl.DeviceIdType.MESH,
            )

        def rdma_b(send_slot):
            return pltpu.make_async_remote_copy(
                src_ref=comm_b.at[send_slot],
                dst_ref=comm_b.at[(send_slot + 1) % 2],
                send_sem=send_b.at[send_slot],
                recv_sem=recv_b.at[(send_slot + 1) % 2],
                device_id=(left,), device_id_type=pl.DeviceIdType.MESH,
            )

        def credit(sem, to):
            pl.semaphore_signal(
                sem, inc=1, device_id=(to,),
                device_id_type=pl.DeviceIdType.MESH,
            )

        def subring(j, carry):
            cols_a = j * NT
            cols_b = half + j * NT
            load_w(cols_a, wahi, walo)
            load_w(cols_b, wbhi, wblo)

            load_x((d - 1) % N_DEV)
            mm(tmpa, wahi, walo)
            comm_a[0] = tmpa[...].astype(jnp.bfloat16)

            @pl.when(j >= 1)
            def _():
                pl.semaphore_wait(credit_a, 1)
            a0 = rdma_a(0)
            a0.start()

            load_x((d + 1) % N_DEV)
            mm(tmpb, wbhi, wblo)
            comm_b[0] = tmpb[...].astype(jnp.bfloat16)

            @pl.when(j >= 1)
            def _():
                pl.semaphore_wait(credit_b, 1)
            b0 = rdma_b(0)
            b0.start()

            load_x((d - 2) % N_DEV)
            mm(tmpa, wahi, walo)
            load_x((d + 2) % N_DEV)
            mm(tmpb, wbhi, wblo)

            a0.wait()
            credit(credit_a, left)
            comm_a[1] = (comm_a[1].astype(jnp.float32)
                         + tmpa[...]).astype(jnp.bfloat16)
            pl.semaphore_wait(credit_a, 1)
            a1 = rdma_a(1)
            a1.start()

            b0.wait()
            credit(credit_b, right)
            comm_b[1] = (comm_b[1].astype(jnp.float32)
                         + tmpb[...]).astype(jnp.bfloat16)
            pl.semaphore_wait(credit_b, 1)
            b1 = rdma_b(1)
            b1.start()

            load_x((d - 3) % N_DEV)
            mm(tmpa, wahi, walo)
            load_x((d + 3) % N_DEV)
            mm(tmpb, wbhi, wblo)

            a1.wait()
            credit(credit_a, left)
            comm_a[0] = (comm_a[0].astype(jnp.float32)
                         + tmpa[...]).astype(jnp.bfloat16)
            pl.semaphore_wait(credit_a, 1)
            a2 = rdma_a(0)
            a2.start()

            b1.wait()
            credit(credit_b, right)
            comm_b[0] = (comm_b[0].astype(jnp.float32)
                         + tmpb[...]).astype(jnp.bfloat16)
            pl.semaphore_wait(credit_b, 1)
            b2 = rdma_b(0)
            b2.start()

            load_x(d)
            mm(tmpa, wahi, walo)
            mm(tmpb, wbhi, wblo)

            a2.wait()
            tmpa[...] = jnp.maximum(comm_a[1].astype(jnp.float32)
                                    + tmpa[...], 0.0)
            amax_smem[0] = jnp.maximum(amax_smem[0], jnp.max(tmpa[...]))
            cp = pltpu.make_async_copy(
                tmpa, out_hbm.at[:, pl.ds(cols_a, NT)], sem_out)
            cp.start()
            cp.wait()

            b2.wait()
            tmpb[...] = jnp.maximum(comm_b[1].astype(jnp.float32)
                                    + tmpb[...], 0.0)
            amax_smem[0] = jnp.maximum(amax_smem[0], jnp.max(tmpb[...]))
            cp = pltpu.make_async_copy(
                tmpb, out_hbm.at[:, pl.ds(cols_b, NT)], sem_out)
            cp.start()
            cp.wait()

            @pl.when(j < nb - 1)
            def _():
                credit(credit_a, left)
                credit(credit_b, right)
            return carry

        lax.fori_loop(0, nb, subring, 0)

        amax_send[...] = jnp.full((1, 128), amax_smem[0], jnp.float32)
        sends = []
        for off in (1, 2, 3):
            p = (d + off) % N_DEV
            s = pltpu.make_async_remote_copy(
                src_ref=amax_send,
                dst_ref=amax_all.at[d],
                send_sem=amax_ssems.at[off],
                recv_sem=amax_rsems.at[d],
                device_id=(p,), device_id_type=pl.DeviceIdType.MESH,
            )
            s.start()
            sends.append(s)
        for off in (1, 2, 3):
            p = (d + off) % N_DEV
            r = pltpu.make_async_remote_copy(
                src_ref=amax_send,
                dst_ref=amax_all.at[p],
                send_sem=amax_ssems.at[0],
                recv_sem=amax_rsems.at[p],
                device_id=(p,), device_id_type=pl.DeviceIdType.MESH,
            )
            r.wait_recv()
        for s in sends:
            s.wait_send()

        g = jnp.maximum(jnp.maximum(amax_smem[0], jnp.max(amax_all[...])),
                        1e-20)
        scale = g / 127.0
        recip = 127.0 / g

        def quant_tile(j, carry):
            ncols = pl.ds(j * NT, NT)
            cp = pltpu.make_async_copy(out_hbm.at[:, ncols], tmpa, sem_out)
            cp.start()
            cp.wait()
            q = jnp.clip(jnp.round(tmpa[...] * recip), -127.0, 127.0)
            tmpb[...] = q * scale
            cp = pltpu.make_async_copy(tmpb, out_hbm.at[:, ncols],
                                       sem_out)
            cp.start()
            cp.wait()
            return carry

        lax.fori_loop(0, n // NT, quant_tile, 0)

    return pl.pallas_call(
        body,
        out_shape=jax.ShapeDtypeStruct((m_blk, n), jnp.float32),
        in_specs=[
            pl.BlockSpec(memory_space=pl.ANY),
            pl.BlockSpec(memory_space=pl.ANY),
        ],
        out_specs=pl.BlockSpec(memory_space=pl.ANY),
        scratch_shapes=[
            pltpu.VMEM((m_blk, k_sh), jnp.float32),
            pltpu.VMEM((m_blk, k_sh), jnp.bfloat16),
            pltpu.VMEM((m_blk, k_sh), jnp.bfloat16),
            pltpu.VMEM((k_sh, NT), jnp.bfloat16),
            pltpu.VMEM((k_sh, NT), jnp.bfloat16),
            pltpu.VMEM((k_sh, NT), jnp.bfloat16),
            pltpu.VMEM((k_sh, NT), jnp.bfloat16),
            pltpu.VMEM((m_blk, NT), jnp.float32),
            pltpu.VMEM((m_blk, NT), jnp.float32),
            pltpu.VMEM((2, m_blk, NT), jnp.bfloat16),
            pltpu.VMEM((2, m_blk, NT), jnp.bfloat16),
            pltpu.SemaphoreType.DMA,
            pltpu.SemaphoreType.DMA,
            pltpu.SemaphoreType.DMA,
            pltpu.SemaphoreType.DMA((2,)),
            pltpu.SemaphoreType.DMA((2,)),
            pltpu.SemaphoreType.DMA((2,)),
            pltpu.SemaphoreType.DMA((2,)),
            pltpu.SemaphoreType.REGULAR,
            pltpu.SemaphoreType.REGULAR,
            pltpu.VMEM((1, 128), jnp.float32),
            pltpu.VMEM((N_DEV, 1, 128), jnp.float32),
            pltpu.SemaphoreType.DMA((N_DEV,)),
            pltpu.SemaphoreType.DMA((N_DEV,)),
            pltpu.SMEM((1,), jnp.float32),
        ],
        compiler_params=pltpu.CompilerParams(
            collective_id=0,
            vmem_limit_bytes=50 * 1024 * 1024,
        ),
    )(x, w_mat)


# device time: 472433 ns/iter; 3.6963x vs baseline; 1.0098x over previous
import jax
import jax.numpy as jnp
from jax import lax
from jax.experimental import pallas as pl
from jax.experimental.pallas import tpu as pltpu

N_DEV = 4
NT = 1024


def kernel(x, w_mat):
    m, k_sh = x.shape
    _, n = w_mat.shape
    m_blk = m // N_DEV
    half = n // 2
    nb = half // NT

    bdot = lambda a, b: jnp.dot(a, b, preferred_element_type=jnp.float32)

    def body(x_hbm, w_hbm, out_hbm, stage, xhi, xlo, wahi, walo, wbhi, wblo,
             tmpa, tmpb, qbuf, comm_a, comm_b,
             sem_x, sem_w, sem_out, sem_qin, sem_qout,
             send_a, recv_a, send_b, recv_b, credit_a, credit_b,
             amax_send, amax_all, amax_ssems, amax_rsems, amax_smem):
        d = lax.axis_index("i")
        left = (d - 1) % N_DEV
        right = (d + 1) % N_DEV

        amax_smem[0] = 0.0
        amax_all[...] = jnp.zeros((N_DEV, 1, 128), jnp.float32)

        barrier_sem = pltpu.get_barrier_semaphore()
        for nbr in [left, right]:
            pl.semaphore_signal(
                barrier_sem, inc=1,
                device_id=(nbr,), device_id_type=pl.DeviceIdType.MESH,
            )
        pl.semaphore_wait(barrier_sem, 2)

        def split(src_f32, hi, lo):
            h = src_f32.astype(jnp.bfloat16)
            hi[...] = h
            lo[...] = (src_f32 - h.astype(jnp.float32)).astype(jnp.bfloat16)

        def load_x(c):
            cp = pltpu.make_async_copy(
                x_hbm.at[pl.ds(c * m_blk, m_blk), :], stage, sem_x)
            cp.start()
            cp.wait()
            split(stage[...], xhi, xlo)

        def load_w(col_start, whi, wlo):
            cp = pltpu.make_async_copy(
                w_hbm.at[:, pl.ds(col_start, NT)], stage, sem_w)
            cp.start()
            cp.wait()
            split(stage[...], whi, wlo)

        def mm(acc, whi, wlo):
            acc[...] = bdot(xhi[...], whi[...])
            acc[...] = acc[...] + bdot(xhi[...], wlo[...])
            acc[...] = acc[...] + bdot(xlo[...], whi[...])

        def rdma_a(send_slot):
            return pltpu.make_async_remote_copy(
                src_ref=comm_a.at[send_slot],
                dst_ref=comm_a.at[(send_slot + 1) % 2],
                send_sem=send_a.at[send_slot],
                recv_sem=recv_a.at[(send_slot + 1) % 2],
                device_id=(right,), device_id_type=pl.DeviceIdType.MESH,
            )

        def rdma_b(send_slot):
            return pltpu.make_async_remote_copy(
                src_ref=comm_b.at[send_slot],
                dst_ref=comm_b.at[(send_slot + 1) % 2],
                send_sem=send_b.at[send_slot],
                recv_sem=recv_b.at[(send_slot + 1) % 2],
                device_id=(left,), device_id_type=pl.DeviceIdType.MESH,
            )

        def credit(sem, to):
            pl.semaphore_signal(
                sem, inc=1, device_id=(to,),
                device_id_type=pl.DeviceIdType.MESH,
            )

        def subring(j, carry):
            cols_a = j * NT
            cols_b = half + j * NT
            load_w(cols_a, wahi, walo)
            load_w(cols_b, wbhi, wblo)

            load_x((d - 1) % N_DEV)
            mm(tmpa, wahi, walo)
            comm_a[0] = tmpa[...].astype(jnp.bfloat16)

            @pl.when(j >= 1)
            def _():
                pl.semaphore_wait(credit_a, 1)
            a0 = rdma_a(0)
            a0.start()

            load_x((d + 1) % N_DEV)
            mm(tmpb, wbhi, wblo)
            comm_b[0] = tmpb[...].astype(jnp.bfloat16)

            @pl.when(j >= 1)
            def _():
                pl.semaphore_wait(credit_b, 1)
            b0 = rdma_b(0)
            b0.start()

            load_x((d - 2) % N_DEV)
            mm(tmpa, wahi, walo)
            load_x((d + 2) % N_DEV)
            mm(tmpb, wbhi, wblo)

            a0.wait()
            credit(credit_a, left)
            comm_a[1] = (comm_a[1].astype(jnp.float32)
                         + tmpa[...]).astype(jnp.bfloat16)
            pl.semaphore_wait(credit_a, 1)
            a1 = rdma_a(1)
            a1.start()

            b0.wait()
            credit(credit_b, right)
            comm_b[1] = (comm_b[1].astype(jnp.float32)
                         + tmpb[...]).astype(jnp.bfloat16)
            pl.semaphore_wait(credit_b, 1)
            b1 = rdma_b(1)
            b1.start()

            load_x((d - 3) % N_DEV)
            mm(tmpa, wahi, walo)
            load_x((d + 3) % N_DEV)
            mm(tmpb, wbhi, wblo)

            a1.wait()
            credit(credit_a, left)
            comm_a[0] = (comm_a[0].astype(jnp.float32)
                         + tmpa[...]).astype(jnp.bfloat16)
            pl.semaphore_wait(credit_a, 1)
            a2 = rdma_a(0)
            a2.start()

            b1.wait()
            credit(credit_b, right)
            comm_b[0] = (comm_b[0].astype(jnp.float32)
                         + tmpb[...]).astype(jnp.bfloat16)
            pl.semaphore_wait(credit_b, 1)
            b2 = rdma_b(0)
            b2.start()

            load_x(d)
            mm(tmpa, wahi, walo)
            mm(tmpb, wbhi, wblo)

            a2.wait()
            tmpa[...] = jnp.maximum(comm_a[1].astype(jnp.float32)
                                    + tmpa[...], 0.0)
            amax_smem[0] = jnp.maximum(amax_smem[0], jnp.max(tmpa[...]))
            cpa = pltpu.make_async_copy(
                tmpa, out_hbm.at[:, pl.ds(cols_a, NT)], sem_out)
            cpa.start()

            b2.wait()
            tmpb[...] = jnp.maximum(comm_b[1].astype(jnp.float32)
                                    + tmpb[...], 0.0)
            amax_smem[0] = jnp.maximum(amax_smem[0], jnp.max(tmpb[...]))
            cpb = pltpu.make_async_copy(
                tmpb, out_hbm.at[:, pl.ds(cols_b, NT)], sem_out)
            cpb.start()
            cpa.wait()
            cpb.wait()

            @pl.when(j < nb - 1)
            def _():
                credit(credit_a, left)
                credit(credit_b, right)
            return carry

        lax.fori_loop(0, nb, subring, 0)

        amax_send[...] = jnp.full((1, 128), amax_smem[0], jnp.float32)
        sends = []
        for off in (1, 2, 3):
            p = (d + off) % N_DEV
            s = pltpu.make_async_remote_copy(
                src_ref=amax_send,
                dst_ref=amax_all.at[d],
                send_sem=amax_ssems.at[off],
                recv_sem=amax_rsems.at[d],
                device_id=(p,), device_id_type=pl.DeviceIdType.MESH,
            )
            s.start()
            sends.append(s)
        for off in (1, 2, 3):
            p = (d + off) % N_DEV
            r = pltpu.make_async_remote_copy(
                src_ref=amax_send,
                dst_ref=amax_all.at[p],
                send_sem=amax_ssems.at[0],
                recv_sem=amax_rsems.at[p],
                device_id=(p,), device_id_type=pl.DeviceIdType.MESH,
            )
            r.wait_recv()
        for s in sends:
            s.wait_send()

        g = jnp.maximum(jnp.maximum(amax_smem[0], jnp.max(amax_all[...])),
                        1e-20)
        scale = g / 127.0
        recip = 127.0 / g

        ins = [tmpa, stage]
        outs = [tmpb, qbuf]
        nq = n // NT
        in_cps = {}
        out_cps = {}

        def in_cp(jq):
            return pltpu.make_async_copy(
                out_hbm.at[:, pl.ds(jq * NT, NT)], ins[jq % 2],
                sem_qin.at[jq % 2])

        def out_cp(jq):
            return pltpu.make_async_copy(
                outs[jq % 2], out_hbm.at[:, pl.ds(jq * NT, NT)],
                sem_qout.at[jq % 2])

        in_cps[0] = in_cp(0)
        in_cps[0].start()
        for jq in range(nq):
            b = jq % 2
            in_cps[jq].wait()
            if jq < nq - 1:
                in_cps[jq + 1] = in_cp(jq + 1)
                in_cps[jq + 1].start()
            if jq >= 2:
                out_cps[jq - 2].wait()
            q = jnp.clip(jnp.round(ins[b][...] * recip), -127.0, 127.0)
            outs[b][...] = q * scale
            out_cps[jq] = out_cp(jq)
            out_cps[jq].start()
        out_cps[nq - 2].wait()
        out_cps[nq - 1].wait()

    return pl.pallas_call(
        body,
        out_shape=jax.ShapeDtypeStruct((m_blk, n), jnp.float32),
        in_specs=[
            pl.BlockSpec(memory_space=pl.ANY),
            pl.BlockSpec(memory_space=pl.ANY),
        ],
        out_specs=pl.BlockSpec(memory_space=pl.ANY),
        scratch_shapes=[
            pltpu.VMEM((m_blk, k_sh), jnp.float32),
            pltpu.VMEM((m_blk, k_sh), jnp.bfloat16),
            pltpu.VMEM((m_blk, k_sh), jnp.bfloat16),
            pltpu.VMEM((k_sh, NT), jnp.bfloat16),
            pltpu.VMEM((k_sh, NT), jnp.bfloat16),
            pltpu.VMEM((k_sh, NT), jnp.bfloat16),
            pltpu.VMEM((k_sh, NT), jnp.bfloat16),
            pltpu.VMEM((m_blk, NT), jnp.float32),
            pltpu.VMEM((m_blk, NT), jnp.float32),
            pltpu.VMEM((m_blk, NT), jnp.float32),
            pltpu.VMEM((2, m_blk, NT), jnp.bfloat16),
            pltpu.VMEM((2, m_blk, NT), jnp.bfloat16),
            pltpu.SemaphoreType.DMA,
            pltpu.SemaphoreType.DMA,
            pltpu.SemaphoreType.DMA,
            pltpu.SemaphoreType.DMA((2,)),
            pltpu.SemaphoreType.DMA((2,)),
            pltpu.SemaphoreType.DMA((2,)),
            pltpu.SemaphoreType.DMA((2,)),
            pltpu.SemaphoreType.DMA((2,)),
            pltpu.SemaphoreType.DMA((2,)),
            pltpu.SemaphoreType.REGULAR,
            pltpu.SemaphoreType.REGULAR,
            pltpu.VMEM((1, 128), jnp.float32),
            pltpu.VMEM((N_DEV, 1, 128), jnp.float32),
            pltpu.SemaphoreType.DMA((N_DEV,)),
            pltpu.SemaphoreType.DMA((N_DEV,)),
            pltpu.SMEM((1,), jnp.float32),
        ],
        compiler_params=pltpu.CompilerParams(
            collective_id=0,
            vmem_limit_bytes=54 * 1024 * 1024,
        ),
    )(x, w_mat)


# device time: 464505 ns/iter; 3.7594x vs baseline; 1.0171x over previous
import jax
import jax.numpy as jnp
from jax import lax
from jax.experimental import pallas as pl
from jax.experimental.pallas import tpu as pltpu

N_DEV = 4
NT = 1024


def kernel(x, w_mat):
    m, k_sh = x.shape
    _, n = w_mat.shape
    m_blk = m // N_DEV
    half = n // 2
    nb = half // NT

    bdot = lambda a, b: jnp.dot(a, b, preferred_element_type=jnp.float32)

    def body(x_hbm, w_hbm, out_hbm, stage, xhi, xlo, wahi, walo, wbhi, wblo,
             tmpa, tmpb, qbuf, hb0, comm_a, comm_b,
             sem_x, sem_w, sem_out, sem_qin, sem_qout,
             send_a, recv_a, send_b, recv_b, credit_a, credit_b,
             amax_send, amax_all, amax_ssems, amax_rsems, amax_smem):
        d = lax.axis_index("i")
        left = (d - 1) % N_DEV
        right = (d + 1) % N_DEV

        amax_smem[0] = 0.0
        amax_all[...] = jnp.zeros((N_DEV, 1, 128), jnp.float32)

        barrier_sem = pltpu.get_barrier_semaphore()
        for nbr in [left, right]:
            pl.semaphore_signal(
                barrier_sem, inc=1,
                device_id=(nbr,), device_id_type=pl.DeviceIdType.MESH,
            )
        pl.semaphore_wait(barrier_sem, 2)

        def split(src_f32, hi, lo):
            h = src_f32.astype(jnp.bfloat16)
            hi[...] = h
            lo[...] = (src_f32 - h.astype(jnp.float32)).astype(jnp.bfloat16)

        def load_x(c):
            cp = pltpu.make_async_copy(
                x_hbm.at[pl.ds(c * m_blk, m_blk), :], stage, sem_x)
            cp.start()
            cp.wait()
            split(stage[...], xhi, xlo)

        def load_w(col_start, whi, wlo):
            cp = pltpu.make_async_copy(
                w_hbm.at[:, pl.ds(col_start, NT)], stage, sem_w)
            cp.start()
            cp.wait()
            split(stage[...], whi, wlo)

        def mm(acc, whi, wlo):
            acc[...] = bdot(xhi[...], whi[...])
            acc[...] = acc[...] + bdot(xhi[...], wlo[...])
            acc[...] = acc[...] + bdot(xlo[...], whi[...])

        def rdma_a(send_slot):
            return pltpu.make_async_remote_copy(
                src_ref=comm_a.at[send_slot],
                dst_ref=comm_a.at[(send_slot + 1) % 2],
                send_sem=send_a.at[send_slot],
                recv_sem=recv_a.at[(send_slot + 1) % 2],
                device_id=(right,), device_id_type=pl.DeviceIdType.MESH,
            )

        def rdma_b(send_slot):
            return pltpu.make_async_remote_copy(
                src_ref=comm_b.at[send_slot],
                dst_ref=comm_b.at[(send_slot + 1) % 2],
                send_sem=send_b.at[send_slot],
                recv_sem=recv_b.at[(send_slot + 1) % 2],
                device_id=(left,), device_id_type=pl.DeviceIdType.MESH,
            )

        def credit(sem, to):
            pl.semaphore_signal(
                sem, inc=1, device_id=(to,),
                device_id_type=pl.DeviceIdType.MESH,
            )

        load_w(0, wahi, walo)
        load_x((d - 1) % N_DEV)
        mm(qbuf, wahi, walo)
        load_w(half, wbhi, wblo)
        load_x((d + 1) % N_DEV)
        mm(hb0, wbhi, wblo)

        def subring(j, carry):
            cols_a = j * NT
            cols_b = half + j * NT

            comm_a[0] = qbuf[...].astype(jnp.bfloat16)

            @pl.when(j >= 1)
            def _():
                pl.semaphore_wait(credit_a, 1)
            a0 = rdma_a(0)
            a0.start()

            comm_b[0] = hb0[...].astype(jnp.bfloat16)

            @pl.when(j >= 1)
            def _():
                pl.semaphore_wait(credit_b, 1)
            b0 = rdma_b(0)
            b0.start()

            load_x((d - 2) % N_DEV)
            mm(tmpa, wahi, walo)
            load_x((d + 2) % N_DEV)
            mm(tmpb, wbhi, wblo)

            a0.wait()
            credit(credit_a, left)
            comm_a[1] = (comm_a[1].astype(jnp.float32)
                         + tmpa[...]).astype(jnp.bfloat16)
            pl.semaphore_wait(credit_a, 1)
            a1 = rdma_a(1)
            a1.start()

            b0.wait()
            credit(credit_b, right)
            comm_b[1] = (comm_b[1].astype(jnp.float32)
                         + tmpb[...]).astype(jnp.bfloat16)
            pl.semaphore_wait(credit_b, 1)
            b1 = rdma_b(1)
            b1.start()

            load_x((d - 3) % N_DEV)
            mm(tmpa, wahi, walo)
            load_x((d + 3) % N_DEV)
            mm(tmpb, wbhi, wblo)

            a1.wait()
            credit(credit_a, left)
            comm_a[0] = (comm_a[0].astype(jnp.float32)
                         + tmpa[...]).astype(jnp.bfloat16)
            pl.semaphore_wait(credit_a, 1)
            a2 = rdma_a(0)
            a2.start()

            b1.wait()
            credit(credit_b, right)
            comm_b[0] = (comm_b[0].astype(jnp.float32)
                         + tmpb[...]).astype(jnp.bfloat16)
            pl.semaphore_wait(credit_b, 1)
            b2 = rdma_b(0)
            b2.start()

            load_x(d)
            mm(tmpa, wahi, walo)
            mm(tmpb, wbhi, wblo)

            @pl.when(j < nb - 1)
            def _():
                load_w((j + 1) * NT, wahi, walo)
                load_x((d - 1) % N_DEV)
                mm(qbuf, wahi, walo)
                load_w(half + (j + 1) * NT, wbhi, wblo)
                load_x((d + 1) % N_DEV)
                mm(hb0, wbhi, wblo)

            a2.wait()
            tmpa[...] = jnp.maximum(comm_a[1].astype(jnp.float32)
                                    + tmpa[...], 0.0)
            amax_smem[0] = jnp.maximum(amax_smem[0], jnp.max(tmpa[...]))
            cpa = pltpu.make_async_copy(
                tmpa, out_hbm.at[:, pl.ds(cols_a, NT)], sem_out)
            cpa.start()

            b2.wait()
            tmpb[...] = jnp.maximum(comm_b[1].astype(jnp.float32)
                                    + tmpb[...], 0.0)
            amax_smem[0] = jnp.maximum(amax_smem[0], jnp.max(tmpb[...]))
            cpb = pltpu.make_async_copy(
                tmpb, out_hbm.at[:, pl.ds(cols_b, NT)], sem_out)
            cpb.start()
            cpa.wait()
            cpb.wait()

            @pl.when(j < nb - 1)
            def _():
                credit(credit_a, left)
                credit(credit_b, right)
            return carry

        lax.fori_loop(0, nb, subring, 0)

        amax_send[...] = jnp.full((1, 128), amax_smem[0], jnp.float32)
        sends = []
        for off in (1, 2, 3):
            p = (d + off) % N_DEV
            s = pltpu.make_async_remote_copy(
                src_ref=amax_send,
                dst_ref=amax_all.at[d],
                send_sem=amax_ssems.at[off],
                recv_sem=amax_rsems.at[d],
                device_id=(p,), device_id_type=pl.DeviceIdType.MESH,
            )
            s.start()
            sends.append(s)
        for off in (1, 2, 3):
            p = (d + off) % N_DEV
            r = pltpu.make_async_remote_copy(
                src_ref=amax_send,
                dst_ref=amax_all.at[p],
                send_sem=amax_ssems.at[0],
                recv_sem=amax_rsems.at[p],
                device_id=(p,), device_id_type=pl.DeviceIdType.MESH,
            )
            r.wait_recv()
        for s in sends:
            s.wait_send()

        g = jnp.maximum(jnp.maximum(amax_smem[0], jnp.max(amax_all[...])),
                        1e-20)
        scale = g / 127.0
        recip = 127.0 / g

        ins = [tmpa, stage]
        outs = [tmpb, qbuf]
        nq = n // NT
        in_cps = {}
        out_cps = {}

        def in_cp(jq):
            return pltpu.make_async_copy(
                out_hbm.at[:, pl.ds(jq * NT, NT)], ins[jq % 2],
                sem_qin.at[jq % 2])

        def out_cp(jq):
            return pltpu.make_async_copy(
                outs[jq % 2], out_hbm.at[:, pl.ds(jq * NT, NT)],
                sem_qout.at[jq % 2])

        in_cps[0] = in_cp(0)
        in_cps[0].start()
        for jq in range(nq):
            b = jq % 2
            in_cps[jq].wait()
            if jq < nq - 1:
                in_cps[jq + 1] = in_cp(jq + 1)
                in_cps[jq + 1].start()
            if jq >= 2:
                out_cps[jq - 2].wait()
            q = jnp.clip(jnp.round(ins[b][...] * recip), -127.0, 127.0)
            outs[b][...] = q * scale
            out_cps[jq] = out_cp(jq)
            out_cps[jq].start()
        out_cps[nq - 2].wait()
        out_cps[nq - 1].wait()

    return pl.pallas_call(
        body,
        out_shape=jax.ShapeDtypeStruct((m_blk, n), jnp.float32),
        in_specs=[
            pl.BlockSpec(memory_space=pl.ANY),
            pl.BlockSpec(memory_space=pl.ANY),
        ],
        out_specs=pl.BlockSpec(memory_space=pl.ANY),
        scratch_shapes=[
            pltpu.VMEM((m_blk, k_sh), jnp.float32),
            pltpu.VMEM((m_blk, k_sh), jnp.bfloat16),
            pltpu.VMEM((m_blk, k_sh), jnp.bfloat16),
            pltpu.VMEM((k_sh, NT), jnp.bfloat16),
            pltpu.VMEM((k_sh, NT), jnp.bfloat16),
            pltpu.VMEM((k_sh, NT), jnp.bfloat16),
            pltpu.VMEM((k_sh, NT), jnp.bfloat16),
            pltpu.VMEM((m_blk, NT), jnp.float32),
            pltpu.VMEM((m_blk, NT), jnp.float32),
            pltpu.VMEM((m_blk, NT), jnp.float32),
            pltpu.VMEM((m_blk, NT), jnp.float32),
            pltpu.VMEM((2, m_blk, NT), jnp.bfloat16),
            pltpu.VMEM((2, m_blk, NT), jnp.bfloat16),
            pltpu.SemaphoreType.DMA,
            pltpu.SemaphoreType.DMA,
            pltpu.SemaphoreType.DMA,
            pltpu.SemaphoreType.DMA((2,)),
            pltpu.SemaphoreType.DMA((2,)),
            pltpu.SemaphoreType.DMA((2,)),
            pltpu.SemaphoreType.DMA((2,)),
            pltpu.SemaphoreType.DMA((2,)),
            pltpu.SemaphoreType.DMA((2,)),
            pltpu.SemaphoreType.REGULAR,
            pltpu.SemaphoreType.REGULAR,
            pltpu.VMEM((1, 128), jnp.float32),
            pltpu.VMEM((N_DEV, 1, 128), jnp.float32),
            pltpu.SemaphoreType.DMA((N_DEV,)),
            pltpu.SemaphoreType.DMA((N_DEV,)),
            pltpu.SMEM((1,), jnp.float32),
        ],
        compiler_params=pltpu.CompilerParams(
            collective_id=0,
            vmem_limit_bytes=58 * 1024 * 1024,
        ),
    )(x, w_mat)
